# baseline (speedup 1.0000x reference)
import functools
import math

import jax
import jax.numpy as jnp
from jax import lax
from jax.experimental import pallas as pl
from jax.experimental.pallas import tpu as pltpu
from jax.experimental.pallas import tpu_sc as plsc

N_EDGES = 100000
N_TRIP = 400000
KMAX = 4
NUM_SPH = 7
EMB_IN = 128
TRIP_IN = 32
EMB_CBF = 16
INV_SQRT_2 = 1.0 / math.sqrt(2.0)

E_BLK = 2000
GRID = N_EDGES // E_BLK

_NW = 32
_CH = 128


def _silu(x):
    return x * jax.nn.sigmoid(x)


def _dense_body(m_ref, rad_ref, w1_ref, w2_ref, wd_ref, o_ref):
    x = jnp.dot(m_ref[...], w1_ref[...], preferred_element_type=jnp.float32)
    x = _silu(x)
    r = jnp.dot(rad_ref[...], w2_ref[...], preferred_element_type=jnp.float32)
    x = x * r
    y = jnp.dot(x, wd_ref[...], preferred_element_type=jnp.float32)
    o_ref[...] = _silu(y)


def _tc_dense(m, bases_rad, w1, w2, wd):
    return pl.pallas_call(
        _dense_body,
        grid=(GRID,),
        in_specs=[
            pl.BlockSpec((E_BLK, EMB_IN), lambda i: (i, 0)),
            pl.BlockSpec((E_BLK, 16), lambda i: (i, 0)),
            pl.BlockSpec((EMB_IN, EMB_IN), lambda i: (0, 0)),
            pl.BlockSpec((16, EMB_IN), lambda i: (0, 0)),
            pl.BlockSpec((EMB_IN, TRIP_IN), lambda i: (0, 0)),
        ],
        out_specs=pl.BlockSpec((E_BLK, TRIP_IN), lambda i: (i, 0)),
        out_shape=jax.ShapeDtypeStruct((N_EDGES, TRIP_IN), jnp.float32),
    )(m, bases_rad, w1, w2, wd)


EP = 102400
LB = 1024


def _trip_body(t_ref, sph_ref, rad_ref, wbt_ref, o_ref):
    a_ks = []
    for k in range(KMAX):
        acc = None
        for s in range(NUM_SPH):
            term = (rad_ref[s * EMB_CBF:(s + 1) * EMB_CBF, :]
                    * sph_ref[s * KMAX + k:s * KMAX + k + 1, :])
            acc = term if acc is None else acc + term
        a_ks.append(acc)
    rbf_rows = []
    for c in range(EMB_CBF):
        acc = None
        for k in range(KMAX):
            term = a_ks[k][c:c + 1, :] * t_ref[k * TRIP_IN:(k + 1) * TRIP_IN, :]
            acc = term if acc is None else acc + term
        rbf_rows.append(acc)
    big = jnp.concatenate(rbf_rows, axis=0)
    xt = jnp.dot(wbt_ref[...], big, preferred_element_type=jnp.float32)
    o_ref[...] = xt


def _tc_trip(t_t, sph_t, rad_t, wbt):
    return pl.pallas_call(
        _trip_body,
        grid=(EP // LB,),
        in_specs=[
            pl.BlockSpec((KMAX * TRIP_IN, LB), lambda i: (0, i)),
            pl.BlockSpec((KMAX * NUM_SPH, LB), lambda i: (0, i)),
            pl.BlockSpec((NUM_SPH * EMB_CBF, LB), lambda i: (0, i)),
            pl.BlockSpec((TRIP_IN, EMB_CBF * TRIP_IN), lambda i: (0, 0)),
        ],
        out_specs=pl.BlockSpec((TRIP_IN, LB), lambda i: (0, i)),
        out_shape=jax.ShapeDtypeStruct((TRIP_IN, EP), jnp.float32),
    )(t_t, sph_t, rad_t, wbt)


def _up_body(x_ref, xp_ref, wca_ref, wac_ref, o_ref):
    a = jnp.dot(x_ref[...], wca_ref[...], preferred_element_type=jnp.float32)
    b = jnp.dot(xp_ref[...], wac_ref[...], preferred_element_type=jnp.float32)
    o_ref[...] = (_silu(a) + _silu(b)) * INV_SQRT_2


def _tc_up(x, xp, wca, wac):
    return pl.pallas_call(
        _up_body,
        grid=(GRID,),
        in_specs=[
            pl.BlockSpec((E_BLK, TRIP_IN), lambda i: (i, 0)),
            pl.BlockSpec((E_BLK, TRIP_IN), lambda i: (i, 0)),
            pl.BlockSpec((TRIP_IN, EMB_IN), lambda i: (0, 0)),
            pl.BlockSpec((TRIP_IN, EMB_IN), lambda i: (0, 0)),
        ],
        out_specs=pl.BlockSpec((E_BLK, EMB_IN), lambda i: (i, 0)),
        out_shape=jax.ShapeDtypeStruct((N_EDGES, EMB_IN), jnp.float32),
    )(x, xp, wca, wac)


@functools.lru_cache(maxsize=None)
def _make_sc_gather(n_idx_rows, width, k_grp):
    npw = n_idx_rows // _NW
    ng = npw // k_grp
    assert npw % k_grp == 0 and n_idx_rows % _NW == 0
    mesh = plsc.VectorSubcoreMesh(core_axis_name="c", subcore_axis_name="s")

    @functools.partial(
        pl.kernel,
        out_type=jax.ShapeDtypeStruct((n_idx_rows * _CH, width), jnp.float32),
        mesh=mesh,
        scratch_types=[
            pltpu.VMEM((npw, _CH), jnp.int32),
            pltpu.VMEM((k_grp * _CH, width), jnp.float32),
            pltpu.VMEM((k_grp * _CH, width), jnp.float32),
            pltpu.SemaphoreType.DMA,
            pltpu.SemaphoreType.DMA,
            pltpu.SemaphoreType.DMA,
        ],
        compiler_params=pltpu.CompilerParams(use_tc_tiling_on_sc=False),
    )
    def k(table_hbm, idx_hbm, out_hbm, idx_v, rows0, rows1, gsem, w0, w1):
        wid = lax.axis_index("s") * 2 + lax.axis_index("c")
        base = wid * npw
        pltpu.sync_copy(idx_hbm.at[pl.ds(base, npw)], idx_v)

        def do_group(g, rows_v, wsem, have_prev):
            @pl.when(have_prev)
            def _():
                pltpu.make_async_copy(
                    rows_v, out_hbm.at[pl.ds(0, k_grp * _CH)], wsem).wait()
            c0 = g * k_grp
            for b in range(k_grp):
                pltpu.async_copy(table_hbm.at[idx_v.at[c0 + b]],
                                 rows_v.at[pl.ds(b * _CH, _CH)], gsem)
            for b in range(k_grp):
                pltpu.make_async_copy(table_hbm.at[idx_v.at[c0 + b]],
                                      rows_v.at[pl.ds(b * _CH, _CH)],
                                      gsem).wait()
            pltpu.async_copy(
                rows_v, out_hbm.at[pl.ds((base + c0) * _CH, k_grp * _CH)],
                wsem)

        def pair(i, carry):
            do_group(2 * i, rows0, w0, i > 0)
            do_group(2 * i + 1, rows1, w1, i > 0)
            return carry

        lax.fori_loop(0, ng // 2, pair, 0)
        if ng % 2:
            do_group(ng - 1, rows0, w0, jnp.bool_(ng > 2))
        pltpu.make_async_copy(rows0, out_hbm.at[pl.ds(0, k_grp * _CH)],
                              w0).wait()
        pltpu.make_async_copy(rows1, out_hbm.at[pl.ds(0, k_grp * _CH)],
                              w1).wait()

    return k


def kernel(m, bases_rad, bases_cir_rad, bases_cir_sph, idx_in, idx_out,
           idx_out_agg, id_swap, W_dense_ba, W_mlp_rbf, W_down, W_bilinear,
           W_up_ca, W_up_ac):
    del idx_out, idx_out_agg

    x_ba = _tc_dense(m, bases_rad, W_dense_ba, W_mlp_rbf, W_down)

    idx2d = jnp.pad(idx_in.reshape(N_TRIP // _CH, _CH),
                    ((0, 3200 - N_TRIP // _CH), (0, 0)))
    gather_trip = _make_sc_gather(3200, TRIP_IN, 4)
    t = gather_trip(x_ba, idx2d)
    t128 = t.reshape(EP, KMAX * TRIP_IN)

    padw = ((0, 0), (0, EP - N_EDGES))
    t_t = t128.T
    sph_t = jnp.pad(
        bases_cir_sph.transpose(2, 1, 0).reshape(NUM_SPH * KMAX, N_EDGES), padw)
    rad_t = jnp.pad(
        bases_cir_rad.transpose(2, 1, 0).reshape(NUM_SPH * EMB_CBF, N_EDGES), padw)
    xt = _tc_trip(t_t, sph_t, rad_t, W_bilinear.T)
    x = xt[:, :N_EDGES].T

    idp = jnp.pad(id_swap, (0, EP - N_EDGES)).reshape(800, _CH)
    gather_perm = _make_sc_gather(800, TRIP_IN, 5)
    xp = gather_perm(x, idp)

    return _tc_up(x, xp, W_up_ca, W_up_ac)


# speedup vs baseline: 1.0118x; 1.0118x over previous
import functools
import math

import jax
import jax.numpy as jnp
from jax import lax
from jax.experimental import pallas as pl
from jax.experimental.pallas import tpu as pltpu
from jax.experimental.pallas import tpu_sc as plsc

N_EDGES = 100000
N_TRIP = 400000
KMAX = 4
NUM_SPH = 7
EMB_IN = 128
TRIP_IN = 32
EMB_CBF = 16
INV_SQRT_2 = 1.0 / math.sqrt(2.0)

E_BLK = 2000
GRID = N_EDGES // E_BLK

_NW = 32
_CH = 128


def _silu(x):
    return x * jax.nn.sigmoid(x)


def _dense_body(m_ref, rad_ref, w1_ref, w2_ref, wd_ref, o_ref):
    x = jnp.dot(m_ref[...], w1_ref[...], preferred_element_type=jnp.float32)
    x = _silu(x)
    r = jnp.dot(rad_ref[...], w2_ref[...], preferred_element_type=jnp.float32)
    x = x * r
    y = jnp.dot(x, wd_ref[...], preferred_element_type=jnp.float32)
    o_ref[...] = _silu(y)


def _tc_dense(m, bases_rad, w1, w2, wd):
    return pl.pallas_call(
        _dense_body,
        grid=(GRID,),
        in_specs=[
            pl.BlockSpec((E_BLK, EMB_IN), lambda i: (i, 0)),
            pl.BlockSpec((E_BLK, 16), lambda i: (i, 0)),
            pl.BlockSpec((EMB_IN, EMB_IN), lambda i: (0, 0)),
            pl.BlockSpec((16, EMB_IN), lambda i: (0, 0)),
            pl.BlockSpec((EMB_IN, TRIP_IN), lambda i: (0, 0)),
        ],
        out_specs=pl.BlockSpec((E_BLK, TRIP_IN), lambda i: (i, 0)),
        out_shape=jax.ShapeDtypeStruct((N_EDGES, TRIP_IN), jnp.float32),
    )(m, bases_rad, w1, w2, wd)


EP = 102400
LB = 1024


def _trip_body(t_ref, sph_ref, rad_ref, wbt_ref, o_ref):
    a_ks = []
    for k in range(KMAX):
        acc = None
        for s in range(NUM_SPH):
            term = (rad_ref[s * EMB_CBF:(s + 1) * EMB_CBF, :]
                    * sph_ref[s * KMAX + k:s * KMAX + k + 1, :])
            acc = term if acc is None else acc + term
        a_ks.append(acc)
    rbf_rows = []
    for c in range(EMB_CBF):
        acc = None
        for k in range(KMAX):
            term = a_ks[k][c:c + 1, :] * t_ref[k * TRIP_IN:(k + 1) * TRIP_IN, :]
            acc = term if acc is None else acc + term
        rbf_rows.append(acc)
    big = jnp.concatenate(rbf_rows, axis=0)
    xt = jnp.dot(wbt_ref[...], big, preferred_element_type=jnp.float32)
    o_ref[...] = xt


def _tc_trip(t_t, sph_t, rad_t, wbt):
    return pl.pallas_call(
        _trip_body,
        grid=(EP // LB,),
        in_specs=[
            pl.BlockSpec((KMAX * TRIP_IN, LB), lambda i: (0, i)),
            pl.BlockSpec((KMAX * NUM_SPH, LB), lambda i: (0, i)),
            pl.BlockSpec((NUM_SPH * EMB_CBF, LB), lambda i: (0, i)),
            pl.BlockSpec((TRIP_IN, EMB_CBF * TRIP_IN), lambda i: (0, 0)),
        ],
        out_specs=pl.BlockSpec((TRIP_IN, LB), lambda i: (0, i)),
        out_shape=jax.ShapeDtypeStruct((TRIP_IN, EP), jnp.float32),
    )(t_t, sph_t, rad_t, wbt)


def _up_body(x_ref, xp_ref, wca_ref, wac_ref, o_ref):
    a = jnp.dot(x_ref[...], wca_ref[...], preferred_element_type=jnp.float32)
    b = jnp.dot(xp_ref[...], wac_ref[...], preferred_element_type=jnp.float32)
    o_ref[...] = (_silu(a) + _silu(b)) * INV_SQRT_2


def _tc_up(x, xp, wca, wac):
    return pl.pallas_call(
        _up_body,
        grid=(GRID,),
        in_specs=[
            pl.BlockSpec((E_BLK, TRIP_IN), lambda i: (i, 0)),
            pl.BlockSpec((E_BLK, TRIP_IN), lambda i: (i, 0)),
            pl.BlockSpec((TRIP_IN, EMB_IN), lambda i: (0, 0)),
            pl.BlockSpec((TRIP_IN, EMB_IN), lambda i: (0, 0)),
        ],
        out_specs=pl.BlockSpec((E_BLK, EMB_IN), lambda i: (i, 0)),
        out_shape=jax.ShapeDtypeStruct((N_EDGES, EMB_IN), jnp.float32),
    )(x, xp, wca, wac)


@functools.lru_cache(maxsize=None)
def _make_sc_gather(n_idx_rows, width, npw0, k0, npw1, k1):
    assert 16 * (npw0 + npw1) == n_idx_rows
    assert npw0 % k0 == 0 and npw1 % k1 == 0
    npw_max = max(npw0, npw1)
    k_max = max(k0, k1)
    mesh = plsc.VectorSubcoreMesh(core_axis_name="c", subcore_axis_name="s")

    @functools.partial(
        pl.kernel,
        out_type=jax.ShapeDtypeStruct((n_idx_rows * _CH, width), jnp.float32),
        mesh=mesh,
        scratch_types=[
            pltpu.VMEM((npw_max, _CH), jnp.int32),
            pltpu.VMEM((k_max * _CH, width), jnp.float32),
            pltpu.VMEM((k_max * _CH, width), jnp.float32),
            pltpu.SemaphoreType.DMA,
            pltpu.SemaphoreType.DMA,
            pltpu.SemaphoreType.DMA,
        ],
        compiler_params=pltpu.CompilerParams(use_tc_tiling_on_sc=False),
    )
    def k(table_hbm, idx_hbm, out_hbm, idx_v, rows0, rows1, gsem, w0, w1):
        sc = lax.axis_index("c")
        s = lax.axis_index("s")

        def do_span(base, npw, k_grp):
            ng = npw // k_grp
            pltpu.sync_copy(idx_hbm.at[pl.ds(base, npw)],
                            idx_v.at[pl.ds(0, npw)])

            def do_group(g, rows_v, wsem, have_prev):
                @pl.when(have_prev)
                def _():
                    pltpu.make_async_copy(
                        rows_v.at[pl.ds(0, k_grp * _CH)],
                        out_hbm.at[pl.ds(0, k_grp * _CH)], wsem).wait()
                c0 = g * k_grp
                for b in range(k_grp):
                    pltpu.async_copy(table_hbm.at[idx_v.at[c0 + b]],
                                     rows_v.at[pl.ds(b * _CH, _CH)], gsem)
                for b in range(k_grp):
                    pltpu.make_async_copy(table_hbm.at[idx_v.at[c0 + b]],
                                          rows_v.at[pl.ds(b * _CH, _CH)],
                                          gsem).wait()
                pltpu.async_copy(
                    rows_v.at[pl.ds(0, k_grp * _CH)],
                    out_hbm.at[pl.ds((base + c0) * _CH, k_grp * _CH)], wsem)

            def pair(i, carry):
                do_group(2 * i, rows0, w0, i > 0)
                do_group(2 * i + 1, rows1, w1, i > 0)
                return carry

            lax.fori_loop(0, ng // 2, pair, 0)
            if ng % 2:
                do_group(ng - 1, rows0, w0, jnp.bool_(ng > 2))
            pltpu.make_async_copy(rows0.at[pl.ds(0, k_grp * _CH)],
                                  out_hbm.at[pl.ds(0, k_grp * _CH)],
                                  w0).wait()
            pltpu.make_async_copy(rows1.at[pl.ds(0, k_grp * _CH)],
                                  out_hbm.at[pl.ds(0, k_grp * _CH)],
                                  w1).wait()

        @pl.when(sc == 0)
        def _():
            do_span(s * npw0, npw0, k0)

        @pl.when(sc == 1)
        def _():
            do_span(16 * npw0 + s * npw1, npw1, k1)

    return k


def kernel(m, bases_rad, bases_cir_rad, bases_cir_sph, idx_in, idx_out,
           idx_out_agg, id_swap, W_dense_ba, W_mlp_rbf, W_down, W_bilinear,
           W_up_ca, W_up_ac):
    del idx_out, idx_out_agg

    x_ba = _tc_dense(m, bases_rad, W_dense_ba, W_mlp_rbf, W_down)

    idx2d = jnp.pad(idx_in.reshape(N_TRIP // _CH, _CH),
                    ((0, 3200 - N_TRIP // _CH), (0, 0)))
    gather_trip = _make_sc_gather(3200, TRIP_IN, 144, 4, 56, 4)
    t = gather_trip(x_ba, idx2d)
    t128 = t.reshape(EP, KMAX * TRIP_IN)

    padw = ((0, 0), (0, EP - N_EDGES))
    t_t = t128.T
    sph_t = jnp.pad(
        bases_cir_sph.transpose(2, 1, 0).reshape(NUM_SPH * KMAX, N_EDGES), padw)
    rad_t = jnp.pad(
        bases_cir_rad.transpose(2, 1, 0).reshape(NUM_SPH * EMB_CBF, N_EDGES), padw)
    xt = _tc_trip(t_t, sph_t, rad_t, W_bilinear.T)
    x = xt[:, :N_EDGES].T

    idp = jnp.pad(id_swap, (0, EP - N_EDGES)).reshape(800, _CH)
    gather_perm = _make_sc_gather(800, TRIP_IN, 35, 5, 15, 5)
    xp = gather_perm(x, idp)

    return _tc_up(x, xp, W_up_ca, W_up_ac)
